# Initial kernel scaffold; baseline (speedup 1.0000x reference)
import functools

import jax
import jax.numpy as jnp
from jax import lax
from jax.experimental import pallas as pl
from jax.experimental.pallas import tpu as pltpu
from jax.experimental.pallas import tpu_sc as plsc

VOCAB = 1000000
EMBED_DIM = 64
MAX_SEQ = 512
B, L = 4096, 200

N = B * L
IW = 128
ROWS = N // IW
NW = 32
ROWS_PER_W = ROWS // NW
K = 4
CHUNK = K * IW
STEPS = ROWS_PER_W // K


def _make_kernel():
    mesh = plsc.VectorSubcoreMesh(core_axis_name="c", subcore_axis_name="s")

    @functools.partial(
        pl.kernel,
        mesh=mesh,
        out_type=jax.ShapeDtypeStruct((N, EMBED_DIM), jnp.float32),
        scratch_types=[
            pltpu.VMEM((K, IW), jnp.int32),
            pltpu.VMEM((K, IW), jnp.int32),
            pltpu.VMEM((CHUNK, EMBED_DIM), jnp.float32),
            pltpu.VMEM((CHUNK, EMBED_DIM), jnp.float32),
            pltpu.SemaphoreType.DMA,
            pltpu.SemaphoreType.DMA,
        ],
    )
    def emb_kernel(x_hbm, pos_hbm, tok_hbm, pe_hbm, out_hbm,
                   idxt_v, idxp_v, a_v, b_v, sem_a, sem_b):
        wid = lax.axis_index("s") * 2 + lax.axis_index("c")
        row_base = wid * ROWS_PER_W

        def step(i, carry):
            row_off = row_base + i * K
            pltpu.sync_copy(x_hbm.at[pl.ds(row_off, K)], idxt_v)
            pltpu.sync_copy(pos_hbm.at[pl.ds(row_off, K)], idxp_v)
            cps = []
            for g in range(K):
                cps.append(pltpu.async_copy(
                    tok_hbm.at[idxt_v.at[g]],
                    a_v.at[pl.ds(g * IW, IW)], sem_a))
                cps.append(pltpu.async_copy(
                    pe_hbm.at[idxp_v.at[g]],
                    b_v.at[pl.ds(g * IW, IW)], sem_b))
            for cp in cps:
                cp.wait()

            def add_row(r, c2):
                for c in range(EMBED_DIM // 16):
                    sl = (r, pl.ds(c * 16, 16))
                    a_v[sl] = a_v[sl] + b_v[sl]
                return c2
            lax.fori_loop(0, CHUNK, add_row, 0, unroll=4)

            pltpu.sync_copy(a_v, out_hbm.at[pl.ds(row_off * IW, CHUNK)])
            return carry

        lax.fori_loop(0, STEPS, step, 0)

    return emb_kernel


_emb = _make_kernel()


@jax.jit
def kernel(x, pos, token_table, position_table):
    x2 = x.reshape(ROWS, IW)
    pos2 = pos.reshape(ROWS, IW)
    out = _emb(x2, pos2, token_table, position_table)
    return out.reshape(B, L, EMBED_DIM)


# speedup vs baseline: 1.9030x; 1.9030x over previous
import functools

import jax
import jax.numpy as jnp
from jax import lax
from jax.experimental import pallas as pl
from jax.experimental.pallas import tpu as pltpu
from jax.experimental.pallas import tpu_sc as plsc

VOCAB = 1000000
EMBED_DIM = 64
MAX_SEQ = 512
B, L = 4096, 200

N = B * L
IW = 128
ROWS = N // IW
NW = 32
ROWS_PER_W = ROWS // NW
K = 4
CHUNK = K * IW
STEPS = ROWS_PER_W // K


def _make_kernel():
    mesh = plsc.VectorSubcoreMesh(core_axis_name="c", subcore_axis_name="s")

    @functools.partial(
        pl.kernel,
        mesh=mesh,
        compiler_params=pltpu.CompilerParams(use_tc_tiling_on_sc=False),
        out_type=jax.ShapeDtypeStruct((N, EMBED_DIM), jnp.float32),
        scratch_types=[
            pltpu.VMEM((K, IW), jnp.int32),
            pltpu.VMEM((K, IW), jnp.int32),
            pltpu.VMEM((CHUNK, EMBED_DIM), jnp.float32),
            pltpu.VMEM((CHUNK, EMBED_DIM), jnp.float32),
            pltpu.SemaphoreType.DMA,
            pltpu.SemaphoreType.DMA,
        ],
    )
    def emb_kernel(x_hbm, pos_hbm, tok_hbm, pe_hbm, out_hbm,
                   idxt_v, idxp_v, a_v, b_v, sem_a, sem_b):
        wid = lax.axis_index("s") * 2 + lax.axis_index("c")
        row_base = wid * ROWS_PER_W

        def step(i, carry):
            row_off = row_base + i * K
            pltpu.sync_copy(x_hbm.at[pl.ds(row_off, K)], idxt_v)
            pltpu.sync_copy(pos_hbm.at[pl.ds(row_off, K)], idxp_v)
            cps = []
            for g in range(K):
                cps.append(pltpu.async_copy(
                    tok_hbm.at[idxt_v.at[g]],
                    a_v.at[pl.ds(g * IW, IW)], sem_a))
                cps.append(pltpu.async_copy(
                    pe_hbm.at[idxp_v.at[g]],
                    b_v.at[pl.ds(g * IW, IW)], sem_b))
            for cp in cps:
                cp.wait()

            def add_row(r, c2):
                for c in range(EMBED_DIM // 16):
                    sl = (r, pl.ds(c * 16, 16))
                    a_v[sl] = a_v[sl] + b_v[sl]
                return c2
            lax.fori_loop(0, CHUNK, add_row, 0, unroll=4)

            pltpu.sync_copy(a_v, out_hbm.at[pl.ds(row_off * IW, CHUNK)])
            return carry

        lax.fori_loop(0, STEPS, step, 0)

    return emb_kernel


_emb = _make_kernel()


@jax.jit
def kernel(x, pos, token_table, position_table):
    x2 = x.reshape(ROWS, IW)
    pos2 = pos.reshape(ROWS, IW)
    out = _emb(x2, pos2, token_table, position_table)
    return out.reshape(B, L, EMBED_DIM)


# speedup vs baseline: 2.3107x; 1.2142x over previous
import functools

import jax
import jax.numpy as jnp
from jax import lax
from jax.experimental import pallas as pl
from jax.experimental.pallas import tpu as pltpu
from jax.experimental.pallas import tpu_sc as plsc

VOCAB = 1000000
EMBED_DIM = 64
MAX_SEQ = 512
B, L = 4096, 200

N = B * L
IW = 128
ROWS = N // IW
NW = 32
ROWS_PER_W = ROWS // NW
K = 2
CHUNK = K * IW
STEPS = ROWS_PER_W // K
NBUF = 2


def _make_kernel():
    mesh = plsc.VectorSubcoreMesh(core_axis_name="c", subcore_axis_name="s")

    scratch = []
    for _ in range(NBUF):
        scratch += [
            pltpu.VMEM((K, IW), jnp.int32),
            pltpu.VMEM((K, IW), jnp.int32),
            pltpu.VMEM((CHUNK, EMBED_DIM), jnp.float32),
            pltpu.VMEM((CHUNK, EMBED_DIM), jnp.float32),
            pltpu.SemaphoreType.DMA,
        ]

    @functools.partial(
        pl.kernel,
        mesh=mesh,
        compiler_params=pltpu.CompilerParams(use_tc_tiling_on_sc=False),
        out_type=jax.ShapeDtypeStruct((N, EMBED_DIM), jnp.float32),
        scratch_types=scratch,
    )
    def emb_kernel(x_hbm, pos_hbm, tok_hbm, pe_hbm, out_hbm, *bufs):
        idxt = [bufs[5 * s + 0] for s in range(NBUF)]
        idxp = [bufs[5 * s + 1] for s in range(NBUF)]
        a = [bufs[5 * s + 2] for s in range(NBUF)]
        b = [bufs[5 * s + 3] for s in range(NBUF)]
        sem = [bufs[5 * s + 4] for s in range(NBUF)]

        wid = lax.axis_index("s") * 2 + lax.axis_index("c")
        row_base = wid * ROWS_PER_W

        def issue(c, s):
            row_off = row_base + c * K
            pltpu.sync_copy(x_hbm.at[pl.ds(row_off, K)], idxt[s])
            pltpu.sync_copy(pos_hbm.at[pl.ds(row_off, K)], idxp[s])
            for g in range(K):
                dsl = pl.ds(g * IW, IW)
                pltpu.async_copy(tok_hbm.at[idxt[s].at[g]], a[s].at[dsl], sem[s])
                pltpu.async_copy(pe_hbm.at[idxp[s].at[g]], b[s].at[dsl], sem[s])

        def finish(c, s):
            for g in range(K):
                dsl = pl.ds(g * IW, IW)
                pltpu.make_async_copy(tok_hbm.at[idxt[s].at[g]], a[s].at[dsl], sem[s]).wait()
                pltpu.make_async_copy(pe_hbm.at[idxp[s].at[g]], b[s].at[dsl], sem[s]).wait()

            @pl.loop(0, CHUNK, unroll=8)
            def add_row(r):
                for col in range(EMBED_DIM // 16):
                    sl = (r, pl.ds(col * 16, 16))
                    plsc.addupdate(a[s].at[sl], b[s][sl])

            row_off = row_base + c * K
            pltpu.sync_copy(a[s], out_hbm.at[pl.ds(row_off * IW, CHUNK)])

        issue(0, 0)

        def pair(m, carry):
            issue(2 * m + 1, 1)
            finish(2 * m, 0)

            @pl.when(2 * m + 2 < STEPS)
            def _():
                issue(2 * m + 2, 0)

            finish(2 * m + 1, 1)
            return carry

        lax.fori_loop(0, STEPS // 2, pair, 0)

    return emb_kernel


_emb = _make_kernel()


@jax.jit
def kernel(x, pos, token_table, position_table):
    x2 = x.reshape(ROWS, IW)
    pos2 = pos.reshape(ROWS, IW)
    out = _emb(x2, pos2, token_table, position_table)
    return out.reshape(B, L, EMBED_DIM)


# speedup vs baseline: 2.3240x; 1.0058x over previous
import functools

import jax
import jax.numpy as jnp
from jax import lax
from jax.experimental import pallas as pl
from jax.experimental.pallas import tpu as pltpu
from jax.experimental.pallas import tpu_sc as plsc

VOCAB = 1000000
EMBED_DIM = 64
MAX_SEQ = 512
B, L = 4096, 200

NW = 32
B_PER_W = B // NW
IW = L // 2
NBUF = 2


def _make_kernel():
    mesh = plsc.VectorSubcoreMesh(core_axis_name="c", subcore_axis_name="s")

    scratch = []
    for _ in range(NBUF):
        scratch += [
            pltpu.VMEM((2, IW), jnp.int32),
            pltpu.VMEM((2, IW), jnp.int32),
            pltpu.VMEM((L, EMBED_DIM), jnp.float32),
            pltpu.VMEM((L, EMBED_DIM), jnp.float32),
            pltpu.SemaphoreType.DMA,
        ]

    @functools.partial(
        pl.kernel,
        mesh=mesh,
        compiler_params=pltpu.CompilerParams(use_tc_tiling_on_sc=False),
        out_type=jax.ShapeDtypeStruct((B, L, EMBED_DIM), jnp.float32),
        scratch_types=scratch,
    )
    def emb_kernel(x_hbm, pos_hbm, tok_hbm, pe_hbm, out_hbm, *bufs):
        idxt = [bufs[5 * s + 0] for s in range(NBUF)]
        idxp = [bufs[5 * s + 1] for s in range(NBUF)]
        a = [bufs[5 * s + 2] for s in range(NBUF)]
        b = [bufs[5 * s + 3] for s in range(NBUF)]
        sem = [bufs[5 * s + 4] for s in range(NBUF)]

        wid = lax.axis_index("s") * 2 + lax.axis_index("c")
        b_base = wid * B_PER_W

        def issue(c, s):
            bb = b_base + c
            pltpu.sync_copy(x_hbm.at[bb], idxt[s])
            pltpu.sync_copy(pos_hbm.at[bb], idxp[s])
            for g in range(2):
                dsl = pl.ds(g * IW, IW)
                pltpu.async_copy(tok_hbm.at[idxt[s].at[g]], a[s].at[dsl], sem[s])
                pltpu.async_copy(pe_hbm.at[idxp[s].at[g]], b[s].at[dsl], sem[s])

        def finish(c, s):
            for g in range(2):
                dsl = pl.ds(g * IW, IW)
                pltpu.make_async_copy(tok_hbm.at[idxt[s].at[g]], a[s].at[dsl], sem[s]).wait()
                pltpu.make_async_copy(pe_hbm.at[idxp[s].at[g]], b[s].at[dsl], sem[s]).wait()

            @pl.loop(0, L, unroll=8)
            def add_row(r):
                for col in range(EMBED_DIM // 16):
                    sl = (r, pl.ds(col * 16, 16))
                    plsc.addupdate(a[s].at[sl], b[s][sl])

            pltpu.sync_copy(a[s], out_hbm.at[b_base + c])

        issue(0, 0)

        def pair(m, carry):
            issue(2 * m + 1, 1)
            finish(2 * m, 0)

            @pl.when(2 * m + 2 < B_PER_W)
            def _():
                issue(2 * m + 2, 0)

            finish(2 * m + 1, 1)
            return carry

        lax.fori_loop(0, B_PER_W // 2, pair, 0)

    return emb_kernel


_emb = _make_kernel()


@jax.jit
def kernel(x, pos, token_table, position_table):
    x3 = x.reshape(B, 2, IW)
    pos3 = pos.reshape(B, 2, IW)
    return _emb(x3, pos3, token_table, position_table)
